# baseline (speedup 1.0000x reference)
import functools

import jax
import jax.numpy as jnp
from jax import lax
from jax.experimental import pallas as pl
from jax.experimental.pallas import tpu as pltpu
from jax.experimental.pallas import tpu_sc as plsc

N = 10000
E = 320000
F = 128
NP = 10240
NC, NS = 2, 16
NW = NC * NS
EPT1 = E // NW
EPT2 = E // NS
K = 80
NCH = EPT2 // K
SEG = 25
RPT = NP // NS


@functools.cache
def _sc_kernels():
  mesh = plsc.VectorSubcoreMesh(
      core_axis_name="c", subcore_axis_name="s", num_cores=NC,
      num_subcores=NS)

  @functools.partial(
      pl.kernel,
      out_type=jax.ShapeDtypeStruct((NC, 2, NP), jnp.float32),
      mesh=mesh,
      compiler_params=pltpu.CompilerParams(use_tc_tiling_on_sc=False, needs_layout_passes=False),
      scratch_types=[
          pltpu.VMEM((EPT1,), jnp.int32),
          pltpu.VMEM((EPT1,), jnp.int32),
          pltpu.VMEM((EPT1,), jnp.float32),
          pltpu.VMEM((NP,), jnp.float32),
          pltpu.VMEM((NP,), jnp.float32),
          pltpu.VMEM((NS, RPT), jnp.float32),
          pltpu.VMEM((RPT,), jnp.float32),
          pltpu.VMEM_SHARED((NS, 2, NP), jnp.float32),
      ],
  )
  def deg_kernel(ei_hbm, w_hbm, out_hbm, row_v, col_v, w_v, ho_v, hi_v,
                 gb_v, rs_v, sh):
    c = lax.axis_index("c")
    s = lax.axis_index("s")
    base = (c * NS + s) * EPT1
    pltpu.sync_copy(ei_hbm.at[0, pl.ds(base, EPT1)], row_v)
    pltpu.sync_copy(ei_hbm.at[1, pl.ds(base, EPT1)], col_v)
    pltpu.sync_copy(w_hbm.at[pl.ds(base, EPT1)], w_v)

    zero = jnp.zeros((16,), jnp.float32)

    def zbody(i, _):
      ho_v[pl.ds(i * 16, 16)] = zero
      hi_v[pl.ds(i * 16, 16)] = zero
      return 0
    lax.fori_loop(0, NP // 16, zbody, 0)

    def ebody(i, _):
      r16 = row_v[pl.ds(i * 16, 16)]
      c16 = col_v[pl.ds(i * 16, 16)]
      w16 = w_v[pl.ds(i * 16, 16)]
      plsc.addupdate_scatter(ho_v, [r16], w16)
      plsc.addupdate_scatter(hi_v, [c16], w16)
      return 0
    lax.fori_loop(0, EPT1 // 16, ebody, 0)

    pltpu.sync_copy(ho_v, sh.at[s, 0])
    pltpu.sync_copy(hi_v, sh.at[s, 1])
    plsc.subcore_barrier()

    for d in range(2):
      pltpu.sync_copy(sh.at[:, d, pl.ds(s * RPT, RPT)], gb_v)

      def rbody(i, _):
        acc = gb_v[0, pl.ds(i * 16, 16)]
        for t in range(1, NS):
          acc = acc + gb_v[t, pl.ds(i * 16, 16)]
        rs_v[pl.ds(i * 16, 16)] = acc
        return 0
      lax.fori_loop(0, RPT // 16, rbody, 0)
      pltpu.sync_copy(rs_v, out_hbm.at[c, d, pl.ds(s * RPT, RPT)])

  @functools.partial(
      pl.kernel,
      out_type=jax.ShapeDtypeStruct((NC, NP, F), jnp.float32),
      mesh=mesh,
      compiler_params=pltpu.CompilerParams(use_tc_tiling_on_sc=False, needs_layout_passes=False),
      scratch_types=[
          pltpu.VMEM((3, SEG, K), jnp.int32),
          pltpu.VMEM((NP // F, F), jnp.float32),
          pltpu.VMEM((SEG * K,), jnp.float32),
          pltpu.VMEM((3, K, F), jnp.float32),
          pltpu.VMEM_SHARED((NP, F), jnp.float32),
          pltpu.SemaphoreType.DMA((3,)),
          pltpu.SemaphoreType.DMA((3,)),
      ],
  )
  def prop_kernel(x_hbm, ed_hbm, degp_hbm, out_hbm,
                  ed_v, inv_v, norms_v, rows_v, acc_sh, gsem, ssem):
    c = lax.axis_index("c")
    s = lax.axis_index("s")

    pltpu.sync_copy(degp_hbm.at[0, c], inv_v)
    pltpu.sync_copy(degp_hbm.at[1, c], rows_v.at[0])

    def ibody(r, _):
      for f in range(F // 16):
        dg = inv_v[r, pl.ds(f * 16, 16)] + rows_v[0, r, pl.ds(f * 16, 16)]
        inv_v[r, pl.ds(f * 16, 16)] = jnp.where(dg > 0.0, 1.0 / dg, 0.0)
      return 0
    lax.fori_loop(0, NP // F, ibody, 0)

    zero = jnp.zeros((16,), jnp.float32)

    def zrow(r, _):
      for f in range(F // 16):
        rows_v[0, r, pl.ds(f * 16, 16)] = zero
      return 0
    lax.fori_loop(0, K, zrow, 0)
    for kk in range(RPT // K):
      pltpu.sync_copy(rows_v.at[0], acc_sh.at[pl.ds(s * RPT + kk * K, K)])
    plsc.subcore_barrier()

    def segment(g, _):
      pltpu.sync_copy(ed_hbm.at[c, s, :, pl.ds(g * SEG, SEG)], ed_v)

      gd = [None] * SEG
      sd = [None] * SEG
      gd[0] = pltpu.async_copy(
          x_hbm.at[ed_v.at[0, 0]], rows_v.at[0], gsem.at[0])
      gd[1] = pltpu.async_copy(
          x_hbm.at[ed_v.at[0, 1]], rows_v.at[1], gsem.at[1])

      def nbody(i, _):
        tq = i // (K // 16)
        to = (i % (K // 16)) * 16
        s16 = ed_v[0, tq, pl.ds(to, 16)]
        w16 = plsc.bitcast(ed_v[2, tq, pl.ds(to, 16)], jnp.float32)
        g16 = plsc.load_gather(
            inv_v, [jax.lax.shift_right_logical(s16, 7),
                    jax.lax.bitwise_and(s16, 127)])
        norms_v[pl.ds(i * 16, 16)] = w16 * g16
        return 0
      lax.fori_loop(0, SEG * K // 16, nbody, 0)

      for t in range(SEG):
        p = t % 3
        if t + 2 < SEG:
          if t >= 1:
            sd[t - 1].wait()
          gd[t + 2] = pltpu.async_copy(
              x_hbm.at[ed_v.at[0, t + 2]], rows_v.at[(t + 2) % 3],
              gsem.at[(t + 2) % 3])
        gd[t].wait()

        def rscale(rr, _):
          for u in range(8):
            r = rr * 8 + u
            nb = plsc.load_gather(
                norms_v, [jnp.full((16,), t * K + r, jnp.int32)])
            for f in range(F // 16):
              rows_v[p, r, pl.ds(f * 16, 16)] = (
                  rows_v[p, r, pl.ds(f * 16, 16)] * nb)
          return 0
        lax.fori_loop(0, K // 8, rscale, 0)

        sd[t] = pltpu.async_copy(
            rows_v.at[p], acc_sh.at[ed_v.at[1, t]], ssem.at[p], add=True)
      sd[SEG - 3].wait()
      sd[SEG - 2].wait()
      sd[SEG - 1].wait()
      return 0
    lax.fori_loop(0, NCH // SEG, segment, 0)
    plsc.subcore_barrier()

    pltpu.sync_copy(acc_sh.at[pl.ds(s * RPT, RPT)],
                    out_hbm.at[c, pl.ds(s * RPT, RPT)])

  return deg_kernel, prop_kernel


BM = 1000


def _head_body(xab_ref, wz_ref, wh_ref, bz_ref, bh_ref, wl_ref, bl_ref,
               o_ref):
  xab = xab_ref[...]
  z = jax.nn.sigmoid(
      jnp.dot(xab, wz_ref[...], preferred_element_type=jnp.float32)
      + bz_ref[...])
  ht = jnp.tanh(
      jnp.dot(xab, wh_ref[...], preferred_element_type=jnp.float32)
      + bh_ref[...])
  hn = jax.nn.relu((1.0 - z) * ht)
  o_ref[...] = (
      jnp.dot(hn, wl_ref[...], preferred_element_type=jnp.float32)
      + bl_ref[...])


_head = pl.pallas_call(
    _head_body,
    grid=(N // BM,),
    in_specs=[
        pl.BlockSpec((BM, 3 * F), lambda i: (i, 0)),
        pl.BlockSpec((3 * F, F), lambda i: (0, 0)),
        pl.BlockSpec((3 * F, F), lambda i: (0, 0)),
        pl.BlockSpec((1, F), lambda i: (0, 0)),
        pl.BlockSpec((1, F), lambda i: (0, 0)),
        pl.BlockSpec((F, 8), lambda i: (0, 0)),
        pl.BlockSpec((1, 8), lambda i: (0, 0)),
    ],
    out_specs=pl.BlockSpec((BM, 8), lambda i: (i, 0)),
    out_shape=jax.ShapeDtypeStruct((N, 8), jnp.float32),
)


def kernel(x, edge_index, edge_weight, W_z, b_z, W_r, b_r, W_h, b_h,
           W_lin, b_lin):
  deg_kernel, prop_kernel = _sc_kernels()
  row = edge_index[0]
  col = edge_index[1]

  degp = deg_kernel(edge_index, edge_weight)

  src_all = jnp.stack([row, col]).reshape(NC, 1, NS, NCH, K)
  dst_all = jnp.stack([col, row]).reshape(NC, 1, NS, NCH, K)
  wbits = jnp.broadcast_to(
      jax.lax.bitcast_convert_type(edge_weight, jnp.int32).reshape(
          1, 1, NS, NCH, K), (NC, 1, NS, NCH, K))
  ed_all = jnp.concatenate([src_all, dst_all, wbits], axis=1)
  ed_all = jnp.transpose(ed_all, (0, 2, 1, 3, 4))
  ab = prop_kernel(x, ed_all, degp.reshape(NC, 2, NP // F, F))

  xab = jnp.concatenate([x, ab[0, :N], ab[1, :N]], axis=1)

  def mats(W):
    return jnp.concatenate([W[0, 0, :F] + W[1, 0, :F], W[0, 1, :F],
                            W[1, 1, :F]], axis=0)

  wl8 = jnp.concatenate([W_lin, jnp.zeros((F, 7), jnp.float32)], axis=1)
  bl8 = jnp.broadcast_to(b_lin.reshape(1, 1), (1, 8))
  out8 = _head(xab, mats(W_z), mats(W_h), b_z.reshape(1, F),
               b_h.reshape(1, F), wl8, bl8)
  return out8[:, :1]


# speedup vs baseline: 1.0072x; 1.0072x over previous
import functools

import jax
import jax.numpy as jnp
from jax import lax
from jax.experimental import pallas as pl
from jax.experimental.pallas import tpu as pltpu
from jax.experimental.pallas import tpu_sc as plsc

N = 10000
E = 320000
F = 128
NP = 10240
NC, NS = 2, 16
NW = NC * NS
EPT1 = E // NW
EPT2 = E // NS
K = 80
NCH = EPT2 // K
SEG = 25
RPT = NP // NS


@functools.cache
def _sc_kernels():
  mesh = plsc.VectorSubcoreMesh(
      core_axis_name="c", subcore_axis_name="s", num_cores=NC,
      num_subcores=NS)

  @functools.partial(
      pl.kernel,
      out_type=jax.ShapeDtypeStruct((NC, 2, NP), jnp.float32),
      mesh=mesh,
      compiler_params=pltpu.CompilerParams(use_tc_tiling_on_sc=False, needs_layout_passes=False),
      scratch_types=[
          pltpu.VMEM((EPT1,), jnp.int32),
          pltpu.VMEM((EPT1,), jnp.int32),
          pltpu.VMEM((EPT1,), jnp.float32),
          pltpu.VMEM((NP,), jnp.float32),
          pltpu.VMEM((NP,), jnp.float32),
          pltpu.VMEM((NS, RPT), jnp.float32),
          pltpu.VMEM((RPT,), jnp.float32),
          pltpu.VMEM_SHARED((NS, 2, NP), jnp.float32),
      ],
  )
  def deg_kernel(ei_hbm, w_hbm, out_hbm, row_v, col_v, w_v, ho_v, hi_v,
                 gb_v, rs_v, sh):
    c = lax.axis_index("c")
    s = lax.axis_index("s")
    base = (c * NS + s) * EPT1
    pltpu.sync_copy(ei_hbm.at[0, pl.ds(base, EPT1)], row_v)
    pltpu.sync_copy(ei_hbm.at[1, pl.ds(base, EPT1)], col_v)
    pltpu.sync_copy(w_hbm.at[pl.ds(base, EPT1)], w_v)

    zero = jnp.zeros((16,), jnp.float32)

    def zbody(i, _):
      ho_v[pl.ds(i * 16, 16)] = zero
      hi_v[pl.ds(i * 16, 16)] = zero
      return 0
    lax.fori_loop(0, NP // 16, zbody, 0)

    def ebody(i, _):
      r16 = row_v[pl.ds(i * 16, 16)]
      c16 = col_v[pl.ds(i * 16, 16)]
      w16 = w_v[pl.ds(i * 16, 16)]
      plsc.addupdate_scatter(ho_v, [r16], w16)
      plsc.addupdate_scatter(hi_v, [c16], w16)
      return 0
    lax.fori_loop(0, EPT1 // 16, ebody, 0)

    pltpu.sync_copy(ho_v, sh.at[s, 0])
    pltpu.sync_copy(hi_v, sh.at[s, 1])
    plsc.subcore_barrier()

    for d in range(2):
      pltpu.sync_copy(sh.at[:, d, pl.ds(s * RPT, RPT)], gb_v)

      def rbody(i, _):
        acc = gb_v[0, pl.ds(i * 16, 16)]
        for t in range(1, NS):
          acc = acc + gb_v[t, pl.ds(i * 16, 16)]
        rs_v[pl.ds(i * 16, 16)] = acc
        return 0
      lax.fori_loop(0, RPT // 16, rbody, 0)
      pltpu.sync_copy(rs_v, out_hbm.at[c, d, pl.ds(s * RPT, RPT)])

  @functools.partial(
      pl.kernel,
      out_type=jax.ShapeDtypeStruct((NC, NP, F), jnp.float32),
      mesh=mesh,
      compiler_params=pltpu.CompilerParams(use_tc_tiling_on_sc=False, needs_layout_passes=False),
      scratch_types=[
          pltpu.VMEM((3, SEG, K), jnp.int32),
          pltpu.VMEM((NP // F, F), jnp.float32),
          pltpu.VMEM((SEG * K,), jnp.float32),
          pltpu.VMEM((3, K, F), jnp.float32),
          pltpu.VMEM_SHARED((NP, F), jnp.float32),
          pltpu.SemaphoreType.DMA((3,)),
          pltpu.SemaphoreType.DMA((3,)),
      ],
  )
  def prop_kernel(x_hbm, ed_hbm, degp_hbm, out_hbm,
                  ed_v, inv_v, norms_v, rows_v, acc_sh, gsem, ssem):
    c = lax.axis_index("c")
    s = lax.axis_index("s")

    pltpu.sync_copy(degp_hbm.at[0, c], inv_v)
    pltpu.sync_copy(degp_hbm.at[1, c], rows_v.at[0])

    def ibody(r, _):
      for f in range(F // 16):
        dg = inv_v[r, pl.ds(f * 16, 16)] + rows_v[0, r, pl.ds(f * 16, 16)]
        inv_v[r, pl.ds(f * 16, 16)] = jnp.where(dg > 0.0, 1.0 / dg, 0.0)
      return 0
    lax.fori_loop(0, NP // F, ibody, 0)

    zero = jnp.zeros((16,), jnp.float32)

    def zrow(r, _):
      for f in range(F // 16):
        rows_v[0, r, pl.ds(f * 16, 16)] = zero
      return 0
    lax.fori_loop(0, K, zrow, 0)
    for kk in range(RPT // K):
      pltpu.sync_copy(rows_v.at[0], acc_sh.at[pl.ds(s * RPT + kk * K, K)])
    plsc.subcore_barrier()

    def segment(g, _):
      pltpu.sync_copy(ed_hbm.at[c, s, :, pl.ds(g * SEG, SEG)], ed_v)

      gd = [None] * SEG
      sd = [None] * SEG
      gd[0] = pltpu.async_copy(
          x_hbm.at[ed_v.at[0, 0]], rows_v.at[0], gsem.at[0])
      gd[1] = pltpu.async_copy(
          x_hbm.at[ed_v.at[0, 1]], rows_v.at[1], gsem.at[1])

      def nbody(i, _):
        tq = i // (K // 16)
        to = (i % (K // 16)) * 16
        s16 = ed_v[0, tq, pl.ds(to, 16)]
        w16 = plsc.bitcast(ed_v[2, tq, pl.ds(to, 16)], jnp.float32)
        g16 = plsc.load_gather(
            inv_v, [jax.lax.shift_right_logical(s16, 7),
                    jax.lax.bitwise_and(s16, 127)])
        norms_v[pl.ds(i * 16, 16)] = w16 * g16
        return 0
      lax.fori_loop(0, SEG * K // 16, nbody, 0)

      for t in range(SEG):
        p = t % 3
        if t + 2 < SEG:
          if t >= 1:
            sd[t - 1].wait()
          gd[t + 2] = pltpu.async_copy(
              x_hbm.at[ed_v.at[0, t + 2]], rows_v.at[(t + 2) % 3],
              gsem.at[(t + 2) % 3])
        gd[t].wait()

        def rscale(rr, _):
          for u in range(4):
            r = rr * 4 + u
            nb = plsc.load_gather(
                norms_v, [jnp.full((16,), t * K + r, jnp.int32)])
            for f in range(F // 16):
              rows_v[p, r, pl.ds(f * 16, 16)] = (
                  rows_v[p, r, pl.ds(f * 16, 16)] * nb)
          return 0
        lax.fori_loop(0, K // 4, rscale, 0)

        sd[t] = pltpu.async_copy(
            rows_v.at[p], acc_sh.at[ed_v.at[1, t]], ssem.at[p], add=True)
      sd[SEG - 3].wait()
      sd[SEG - 2].wait()
      sd[SEG - 1].wait()
      return 0
    lax.fori_loop(0, NCH // SEG, segment, 0)
    plsc.subcore_barrier()

    pltpu.sync_copy(acc_sh.at[pl.ds(s * RPT, RPT)],
                    out_hbm.at[c, pl.ds(s * RPT, RPT)])

  return deg_kernel, prop_kernel


BM = 1000


def _head_body(xab_ref, wz_ref, wh_ref, bz_ref, bh_ref, wl_ref, bl_ref,
               o_ref):
  xab = xab_ref[...]
  z = jax.nn.sigmoid(
      jnp.dot(xab, wz_ref[...], preferred_element_type=jnp.float32)
      + bz_ref[...])
  ht = jnp.tanh(
      jnp.dot(xab, wh_ref[...], preferred_element_type=jnp.float32)
      + bh_ref[...])
  hn = jax.nn.relu((1.0 - z) * ht)
  o_ref[...] = (
      jnp.dot(hn, wl_ref[...], preferred_element_type=jnp.float32)
      + bl_ref[...])


_head = pl.pallas_call(
    _head_body,
    grid=(N // BM,),
    in_specs=[
        pl.BlockSpec((BM, 3 * F), lambda i: (i, 0)),
        pl.BlockSpec((3 * F, F), lambda i: (0, 0)),
        pl.BlockSpec((3 * F, F), lambda i: (0, 0)),
        pl.BlockSpec((1, F), lambda i: (0, 0)),
        pl.BlockSpec((1, F), lambda i: (0, 0)),
        pl.BlockSpec((F, 8), lambda i: (0, 0)),
        pl.BlockSpec((1, 8), lambda i: (0, 0)),
    ],
    out_specs=pl.BlockSpec((BM, 8), lambda i: (i, 0)),
    out_shape=jax.ShapeDtypeStruct((N, 8), jnp.float32),
)


def kernel(x, edge_index, edge_weight, W_z, b_z, W_r, b_r, W_h, b_h,
           W_lin, b_lin):
  deg_kernel, prop_kernel = _sc_kernels()
  row = edge_index[0]
  col = edge_index[1]

  degp = deg_kernel(edge_index, edge_weight)

  src_all = jnp.stack([row, col]).reshape(NC, 1, NS, NCH, K)
  dst_all = jnp.stack([col, row]).reshape(NC, 1, NS, NCH, K)
  wbits = jnp.broadcast_to(
      jax.lax.bitcast_convert_type(edge_weight, jnp.int32).reshape(
          1, 1, NS, NCH, K), (NC, 1, NS, NCH, K))
  ed_all = jnp.concatenate([src_all, dst_all, wbits], axis=1)
  ed_all = jnp.transpose(ed_all, (0, 2, 1, 3, 4))
  ab = prop_kernel(x, ed_all, degp.reshape(NC, 2, NP // F, F))

  xab = jnp.concatenate([x, ab[0, :N], ab[1, :N]], axis=1)

  def mats(W):
    return jnp.concatenate([W[0, 0, :F] + W[1, 0, :F], W[0, 1, :F],
                            W[1, 1, :F]], axis=0)

  wl8 = jnp.concatenate([W_lin, jnp.zeros((F, 7), jnp.float32)], axis=1)
  bl8 = jnp.broadcast_to(b_lin.reshape(1, 1), (1, 8))
  out8 = _head(xab, mats(W_z), mats(W_h), b_z.reshape(1, F),
               b_h.reshape(1, F), wl8, bl8)
  return out8[:, :1]
